# baseline (speedup 1.0000x reference)
import functools

import jax
import jax.numpy as jnp
from jax import lax
from jax.experimental import pallas as pl
from jax.experimental.pallas import tpu as pltpu
from jax.experimental.pallas import tpu_sc as plsc

_B = 16384
_D = 16
_GRP = 128 // _D
_NC = 2
_NS = 16
_NW = _NC * _NS
_BPW = _B // _NW
_CH = 128
_NCH = _BPW // _CH
_L = 16
_NBLK = _CH // _L
_SHARP = 5.0


def _body(u_emb, v_emb, rec, u, out, gidx, goff, vrows, urows, outv, sem_v, sem_u):
    wid = lax.axis_index("s") * _NC + lax.axis_index("c")
    base = wid * _BPW

    lanes = lax.iota(jnp.int32, 16)

    for i in range(_NCH):
        pltpu.sync_copy(rec.at[pl.ds(base + i * _CH, _CH)], gidx.at[0].at[i])
        pltpu.sync_copy(u.at[pl.ds(base + i * _CH, _CH)], gidx.at[1].at[i])
    for t in range(2):
        for i in range(_NCH):
            for j in range(_CH // _L):
                raw = gidx[t, i, pl.ds(j * _L, _L)]
                goff[t, i, pl.ds(j * _L, _L)] = (raw & (_GRP - 1)) * _D
                gidx[t, i, pl.ds(j * _L, _L)] = lax.shift_right_logical(raw, 3)

    def chunk(i, carry):
        cv = pltpu.async_copy(v_emb.at[gidx.at[0].at[i]], vrows, sem_v)
        cu = pltpu.async_copy(u_emb.at[gidx.at[1].at[i]], urows, sem_u)
        cv.wait()
        cu.wait()

        def blk(j, c2):
            rows_idx = j * _L + lanes
            voff = goff[0, i, pl.ds(j * _L, _L)]
            uoff = goff[1, i, pl.ds(j * _L, _L)]
            acc = jnp.zeros((_L,), jnp.float32)
            for d in range(_D):
                vcol = plsc.load_gather(vrows, [rows_idx, voff + d])
                ucol = plsc.load_gather(urows, [rows_idx, uoff + d])
                acc = acc + vcol * ucol
            sig = 1.0 / (1.0 + jnp.exp(-_SHARP * acc))
            outv[pl.ds(i * _CH + j * _L, _L)] = sig
            return c2

        lax.fori_loop(0, _NBLK, blk, 0)
        return carry

    lax.fori_loop(0, _NCH, chunk, 0)

    pltpu.sync_copy(outv, out.at[pl.ds(base, _BPW)])


def kernel(uEmb, vEmb, rec, u):
    rec = rec.astype(jnp.int32)
    u = u.astype(jnp.int32)
    u_g = jnp.reshape(uEmb, (uEmb.shape[0] // _GRP, 128))
    v_g = jnp.reshape(vEmb, (vEmb.shape[0] // _GRP, 128))
    mesh = plsc.VectorSubcoreMesh(core_axis_name="c", subcore_axis_name="s")
    f = pl.kernel(
        _body,
        mesh=mesh,
        out_type=jax.ShapeDtypeStruct((_B,), jnp.float32),
        scratch_types=[
            pltpu.VMEM((2, _NCH, _CH), jnp.int32),
            pltpu.VMEM((2, _NCH, _CH), jnp.int32),
            pltpu.VMEM((_CH, 128), jnp.float32),
            pltpu.VMEM((_CH, 128), jnp.float32),
            pltpu.VMEM((_BPW,), jnp.float32),
            pltpu.SemaphoreType.DMA,
            pltpu.SemaphoreType.DMA,
        ],
        compiler_params=pltpu.CompilerParams(
            needs_layout_passes=False, use_tc_tiling_on_sc=False),
    )
    return f(u_g, v_g, rec, u)


# speedup vs baseline: 1.0016x; 1.0016x over previous
import functools

import jax
import jax.numpy as jnp
from jax import lax
from jax.experimental import pallas as pl
from jax.experimental.pallas import tpu as pltpu
from jax.experimental.pallas import tpu_sc as plsc

_B = 16384
_D = 16
_GRP = 128 // _D
_NC = 2
_NS = 16
_NW = _NC * _NS
_BPW = _B // _NW
_CH = 128
_NCH = _BPW // _CH
_L = 16
_NBLK = _CH // _L
_SHARP = 5.0


def _body(u_emb, v_emb, rec, u, out, gidx, goff, vrows, urows, outv, sem_v, sem_u):
    wid = lax.axis_index("s") * _NC + lax.axis_index("c")
    base = wid * _BPW

    lanes = lax.iota(jnp.int32, 16)

    for i in range(_NCH):
        pltpu.sync_copy(rec.at[pl.ds(base + i * _CH, _CH)], gidx.at[0].at[i])
        pltpu.sync_copy(u.at[pl.ds(base + i * _CH, _CH)], gidx.at[1].at[i])
    for t in range(2):
        for i in range(_NCH):
            for j in range(_CH // _L):
                raw = gidx[t, i, pl.ds(j * _L, _L)]
                goff[t, i, pl.ds(j * _L, _L)] = (raw & (_GRP - 1)) * _D
                gidx[t, i, pl.ds(j * _L, _L)] = lax.shift_right_logical(raw, 3)

    def chunk(i, carry):
        cv = pltpu.async_copy(v_emb.at[gidx.at[0].at[i]], vrows, sem_v)
        cu = pltpu.async_copy(u_emb.at[gidx.at[1].at[i]], urows, sem_u)
        cv.wait()
        cu.wait()

        def blk(j, c2):
            rows_idx = j * _L + lanes
            voff = goff[0, i, pl.ds(j * _L, _L)]
            uoff = goff[1, i, pl.ds(j * _L, _L)]
            acc = jnp.zeros((_L,), jnp.float32)
            for d in range(_D):
                vcol = plsc.load_gather(vrows, [rows_idx, voff + d])
                ucol = plsc.load_gather(urows, [rows_idx, uoff + d])
                acc = acc + vcol * ucol
            sig = 1.0 / (1.0 + jnp.exp(-_SHARP * acc))
            outv[pl.ds(i * _CH + j * _L, _L)] = sig
            return c2

        lax.fori_loop(0, _NBLK, blk, 0)
        return carry

    lax.fori_loop(0, _NCH, chunk, 0)

    pltpu.sync_copy(outv, out.at[pl.ds(base, _BPW)])


def kernel(uEmb, vEmb, rec, u):
    rec = rec.astype(jnp.int32)
    u = u.astype(jnp.int32)
    u_g = jnp.reshape(uEmb, (uEmb.shape[0] // _GRP, 128))
    v_g = jnp.reshape(vEmb, (vEmb.shape[0] // _GRP, 128))
    mesh = plsc.VectorSubcoreMesh(core_axis_name="c", subcore_axis_name="s")
    f = pl.kernel(
        _body,
        mesh=mesh,
        out_type=jax.ShapeDtypeStruct((_B,), jnp.float32),
        scratch_types=[
            pltpu.VMEM((2, _NCH, _CH), jnp.int32),
            pltpu.VMEM((2, _NCH, _CH), jnp.int32),
            pltpu.VMEM((_CH, 128), jnp.float32),
            pltpu.VMEM((_CH, 128), jnp.float32),
            pltpu.VMEM((_BPW,), jnp.float32),
            pltpu.SemaphoreType.DMA,
            pltpu.SemaphoreType.DMA,
        ],
        compiler_params=pltpu.CompilerParams(
            needs_layout_passes=False, use_tc_tiling_on_sc=True),
    )
    return f(u_g, v_g, rec, u)


# speedup vs baseline: 1.0124x; 1.0108x over previous
import functools

import jax
import jax.numpy as jnp
from jax import lax
from jax.experimental import pallas as pl
from jax.experimental.pallas import tpu as pltpu
from jax.experimental.pallas import tpu_sc as plsc

_B = 16384
_D = 16
_NC = 2
_NS = 16
_NW = _NC * _NS
_BPW = _B // _NW
_CH = 128
_NCH = _BPW // _CH
_L = 16
_NBLK = _BPW // _L
_SHARP = 5.0


def _body(u_emb, v_emb, rec, u, out, recv, uv, vrows, urows, outv, sem_v, sem_u):
    wid = lax.axis_index("s") * _NC + lax.axis_index("c")
    base = wid * _BPW

    for i in range(_NCH):
        pltpu.sync_copy(rec.at[pl.ds(base + i * _CH, _CH)], recv.at[i])
        pltpu.sync_copy(u.at[pl.ds(base + i * _CH, _CH)], uv.at[i])

    copies = []
    for i in range(_NCH):
        copies.append(
            pltpu.async_copy(v_emb.at[recv.at[i]], vrows.at[pl.ds(i * _CH, _CH)], sem_v))
        copies.append(
            pltpu.async_copy(u_emb.at[uv.at[i]], urows.at[pl.ds(i * _CH, _CH)], sem_u))
    for c in copies:
        c.wait()

    lanes = lax.iota(jnp.int32, 16)

    def blk(j, carry):
        rows_idx = j * _L + lanes
        acc = jnp.zeros((_L,), jnp.float32)
        for d in range(_D):
            col = jnp.full((_L,), d, jnp.int32)
            vcol = plsc.load_gather(vrows, [rows_idx, col])
            ucol = plsc.load_gather(urows, [rows_idx, col])
            acc = acc + vcol * ucol
        sig = 1.0 / (1.0 + jnp.exp(-_SHARP * acc))
        outv[pl.ds(j * _L, _L)] = sig
        return carry

    lax.fori_loop(0, _NBLK, blk, 0)

    pltpu.sync_copy(outv, out.at[pl.ds(base, _BPW)])


def kernel(uEmb, vEmb, rec, u):
    rec = rec.astype(jnp.int32)
    u = u.astype(jnp.int32)
    mesh = plsc.VectorSubcoreMesh(core_axis_name="c", subcore_axis_name="s")
    f = pl.kernel(
        _body,
        mesh=mesh,
        out_type=jax.ShapeDtypeStruct((_B,), jnp.float32),
        scratch_types=[
            pltpu.VMEM((_NCH, _CH), jnp.int32),
            pltpu.VMEM((_NCH, _CH), jnp.int32),
            pltpu.VMEM((_BPW, _D), jnp.float32),
            pltpu.VMEM((_BPW, _D), jnp.float32),
            pltpu.VMEM((_BPW,), jnp.float32),
            pltpu.SemaphoreType.DMA,
            pltpu.SemaphoreType.DMA,
        ],
        compiler_params=pltpu.CompilerParams(
            needs_layout_passes=False, use_tc_tiling_on_sc=False),
    )
    return f(uEmb, vEmb, rec, u)
